# Initial kernel scaffold; baseline (speedup 1.0000x reference)
import jax
import jax.numpy as jnp
from jax.experimental import pallas as pl


def kernel(sentences, V, W, b):
    raise NotImplementedError("write your pallas kernel here")


# speedup vs baseline: 1.5339x; 1.5339x over previous
import functools

import jax
import jax.numpy as jnp
from jax import lax
from jax.experimental import pallas as pl
from jax.experimental.pallas import tpu as pltpu
from jax.experimental.pallas import tpu_sc as plsc

B = 1024
L = 50
D = 64
N_LABELS = 128

NUM_CORES = 2
NUM_SUBCORES = 16
NW = NUM_CORES * NUM_SUBCORES
SENT_PER_W = B // NW
IDX_PER_W = SENT_PER_W * L
CHUNK_SENTS = 2
CHUNK_IDX = CHUNK_SENTS * L
NCHUNK = SENT_PER_W // CHUNK_SENTS
LANES = 16
DQ = D // LANES

_mesh = plsc.VectorSubcoreMesh(core_axis_name="c", subcore_axis_name="s")


@functools.partial(
    pl.kernel,
    out_type=jax.ShapeDtypeStruct((B, D), jnp.float32),
    mesh=_mesh,
    scratch_types=[
        pltpu.VMEM((NCHUNK, CHUNK_IDX), jnp.int32),
        pltpu.VMEM((IDX_PER_W, D), jnp.float32),
        pltpu.VMEM((SENT_PER_W, D), jnp.float32),
        pltpu.SemaphoreType.DMA,
    ],
    compiler_params=pltpu.CompilerParams(use_tc_tiling_on_sc=False),
)
def _pool_sc(idx_hbm, v_hbm, out_hbm, idx_v, rows_v, x_v, sem):
    wid = lax.axis_index("s") * NUM_CORES + lax.axis_index("c")
    sent_base = wid * SENT_PER_W

    pltpu.sync_copy(idx_hbm.at[wid], idx_v)

    copies = [
        pltpu.async_copy(
            v_hbm.at[idx_v.at[j]],
            rows_v.at[pl.ds(j * CHUNK_IDX, CHUNK_IDX)],
            sem,
        )
        for j in range(NCHUNK)
    ]
    for c in copies:
        c.wait()

    inv_len = jnp.float32(1.0 / L)

    def sent_body(s, carry):
        def tok_body(t, accs):
            base = s * L + t
            return tuple(
                accs[q] + rows_v[base, pl.ds(q * LANES, LANES)]
                for q in range(DQ)
            )
        accs = lax.fori_loop(
            0, L, tok_body,
            tuple(jnp.zeros((LANES,), jnp.float32) for _ in range(DQ)),
        )
        for q in range(DQ):
            x_v[s, pl.ds(q * LANES, LANES)] = accs[q] * inv_len
        return carry

    lax.fori_loop(0, SENT_PER_W, sent_body, 0)

    pltpu.sync_copy(x_v, out_hbm.at[pl.ds(sent_base, SENT_PER_W)])


def _linear_body(x_ref, w_ref, b_ref, o_ref):
    o_ref[...] = (
        lax.dot_general(
            x_ref[...], w_ref[...], (((1,), (1,)), ((), ())),
            preferred_element_type=jnp.float32,
        )
        + b_ref[...]
    )


_linear_tc = pl.pallas_call(
    _linear_body,
    out_shape=jax.ShapeDtypeStruct((B, N_LABELS), jnp.float32),
)


def kernel(sentences, V, W, b):
    idx = sentences.astype(jnp.int32).reshape(NW, NCHUNK, CHUNK_IDX)
    x = _pool_sc(idx, V)
    return _linear_tc(x, W, b.reshape(1, N_LABELS))
